# baseline (speedup 1.0000x reference)
import functools

import jax
import jax.numpy as jnp
from jax import lax
from jax.experimental import pallas as pl
from jax.experimental.pallas import tpu as pltpu
from jax.experimental.pallas import tpu_sc as plsc

PADDING_IDX = 0
LANES = 16
CHUNK = 16
NQ = 2
QROWS = CHUNK // NQ


def _make_sc_embed(bsz, seq_len, d):
    info = plsc.get_sparse_core_info()
    nw = info.num_cores * info.num_subcores
    nc = info.num_cores
    assert seq_len % (nw * CHUNK) == 0
    seq_per_w = seq_len // nw
    n_chunks = seq_per_w // CHUNK
    n_groups = seq_per_w // LANES

    mesh = plsc.VectorSubcoreMesh(core_axis_name="c", subcore_axis_name="s")

    @functools.partial(
        pl.kernel,
        mesh=mesh,
        out_type=jax.ShapeDtypeStruct((bsz * seq_len, d), jnp.float32),
        scratch_types=[
            pltpu.VMEM((bsz * seq_per_w,), jnp.int32),
            pltpu.VMEM((LANES,), jnp.int32),
        ]
        + [pltpu.VMEM((QROWS, d), jnp.float32) for _ in range(NQ)]
        + [pltpu.SemaphoreType.DMA for _ in range(2 * NQ)],
    )
    def sc_embed(inp_hbm, w_hbm, out_hbm, inp_v, idx_v, *bufsem):
        bufs = bufsem[:NQ]
        semgs = bufsem[NQ : 2 * NQ]
        semws = bufsem[2 * NQ :]
        wid = lax.axis_index("s") * nc + lax.axis_index("c")
        s0 = wid * seq_per_w
        for b in range(bsz):
            pltpu.sync_copy(
                inp_hbm.at[pl.ds(b * seq_len + s0, seq_per_w)],
                inp_v.at[pl.ds(b * seq_per_w, seq_per_w)],
            )
        lane = lax.iota(jnp.int32, LANES)

        def drain_writes(q):
            for _ in range(bsz):
                pltpu.make_async_copy(
                    bufs[q], out_hbm.at[pl.ds(s0, QROWS)], semws[q]
                ).wait()

        def issue_gather(q):
            pltpu.async_copy(
                w_hbm.at[idx_v.at[pl.ds(q * QROWS, QROWS)]], bufs[q], semgs[q]
            )

        def wait_gather(q):
            pltpu.make_async_copy(
                w_hbm.at[idx_v.at[pl.ds(q * QROWS, QROWS)]], bufs[q], semgs[q]
            ).wait()

        def fire_writes(q, row0):
            for b in range(bsz):
                pltpu.async_copy(
                    bufs[q],
                    out_hbm.at[pl.ds(b * seq_len + row0, QROWS)],
                    semws[q],
                )

        idx_v[...] = s0 + 1 + lane
        issue_gather(0)

        def copy_chunk(i, carry):
            @pl.when(i > 0)
            def _():
                drain_writes(1)

            issue_gather(1)
            wait_gather(0)
            fire_writes(0, s0 + i * CHUNK)
            wait_gather(1)
            fire_writes(1, s0 + i * CHUNK + QROWS)
            drain_writes(0)

            @pl.when(i < n_chunks - 1)
            def _():
                idx_v[...] = s0 + (i + 1) * CHUNK + 1 + lane
                issue_gather(0)

            return carry

        lax.fori_loop(0, n_chunks, copy_chunk, 0)
        drain_writes(1)

        def fix_group(b, j):
            tok = inp_v[pl.ds(b * seq_per_w + j * LANES, LANES)]
            has_pad = tok[0] == PADDING_IDX
            for r in range(1, LANES):
                has_pad = has_pad | (tok[r] == PADDING_IDX)

            @pl.when(has_pad)
            def _():
                idx_v[...] = jnp.where(
                    tok != PADDING_IDX, s0 + j * LANES + lane + 1, PADDING_IDX
                )
                for q in range(NQ):
                    pltpu.async_copy(
                        w_hbm.at[idx_v.at[pl.ds(q * QROWS, QROWS)]],
                        bufs[q],
                        semgs[q],
                    ).wait()
                    pltpu.sync_copy(
                        bufs[q],
                        out_hbm.at[
                            pl.ds(b * seq_len + s0 + j * LANES + q * QROWS, QROWS)
                        ],
                    )

        for b in range(bsz):
            lax.fori_loop(0, n_groups, lambda j, c, b=b: (fix_group(b, j), c)[1], 0)

    return sc_embed


def kernel(input, weights):
    bsz, seq_len = input.shape
    d = weights.shape[1]
    sc_embed = _make_sc_embed(bsz, seq_len, d)
    out = sc_embed(input.reshape(-1), weights)
    return out.reshape(bsz, seq_len, d)


# speedup vs baseline: 1.0320x; 1.0320x over previous
import functools

import jax
import jax.numpy as jnp
from jax import lax
from jax.experimental import pallas as pl
from jax.experimental.pallas import tpu as pltpu
from jax.experimental.pallas import tpu_sc as plsc

PADDING_IDX = 0
LANES = 16
CHUNK = 16
NQ = 2
QROWS = CHUNK // NQ


def _make_sc_embed(bsz, seq_len, d):
    info = plsc.get_sparse_core_info()
    nw = info.num_cores * info.num_subcores
    nc = info.num_cores
    assert seq_len % (nw * CHUNK) == 0
    seq_per_w = seq_len // nw
    n_chunks = seq_per_w // CHUNK
    n_groups = seq_per_w // LANES

    mesh = plsc.VectorSubcoreMesh(core_axis_name="c", subcore_axis_name="s")

    @functools.partial(
        pl.kernel,
        mesh=mesh,
        out_type=jax.ShapeDtypeStruct((bsz * seq_len, d), jnp.float32),
        scratch_types=[
            pltpu.VMEM((bsz * seq_per_w,), jnp.int32),
            pltpu.VMEM((LANES,), jnp.int32),
        ]
        + [pltpu.VMEM((QROWS, d), jnp.float32) for _ in range(NQ)]
        + [pltpu.SemaphoreType.DMA for _ in range(2 * NQ + 1)],
    )
    def sc_embed(inp_hbm, w_hbm, out_hbm, inp_v, idx_v, *bufsem):
        bufs = bufsem[:NQ]
        semgs = bufsem[NQ : 2 * NQ]
        semws = bufsem[2 * NQ : 3 * NQ]
        semi = bufsem[3 * NQ]
        wid = lax.axis_index("s") * nc + lax.axis_index("c")
        s0 = wid * seq_per_w
        def stage_inp(start):
            for b in range(bsz):
                op = pltpu.async_copy if start else pltpu.make_async_copy
                h = op(
                    inp_hbm.at[pl.ds(b * seq_len + s0, seq_per_w)],
                    inp_v.at[pl.ds(b * seq_per_w, seq_per_w)],
                    semi,
                )
                if not start:
                    h.wait()

        stage_inp(True)
        lane = lax.iota(jnp.int32, LANES)

        def drain_writes(q):
            for _ in range(bsz):
                pltpu.make_async_copy(
                    bufs[q], out_hbm.at[pl.ds(s0, QROWS)], semws[q]
                ).wait()

        def issue_gather(q):
            pltpu.async_copy(
                w_hbm.at[idx_v.at[pl.ds(q * QROWS, QROWS)]], bufs[q], semgs[q]
            )

        def wait_gather(q):
            pltpu.make_async_copy(
                w_hbm.at[idx_v.at[pl.ds(q * QROWS, QROWS)]], bufs[q], semgs[q]
            ).wait()

        def fire_writes(q, row0):
            for b in range(bsz):
                pltpu.async_copy(
                    bufs[q],
                    out_hbm.at[pl.ds(b * seq_len + row0, QROWS)],
                    semws[q],
                )

        def copy_chunk(i, carry):
            idx_v[...] = s0 + i * CHUNK + 1 + lane
            for q in range(NQ):
                @pl.when(i > 0)
                def _():
                    drain_writes(q)

                issue_gather(q)
                wait_gather(q)
                fire_writes(q, s0 + i * CHUNK + q * QROWS)
            return carry

        lax.fori_loop(0, n_chunks, copy_chunk, 0)
        for q in range(NQ):
            drain_writes(q)
        stage_inp(False)

        def fix_group(b, j):
            tok = inp_v[pl.ds(b * seq_per_w + j * LANES, LANES)]
            has_pad = tok[0] == PADDING_IDX
            for r in range(1, LANES):
                has_pad = has_pad | (tok[r] == PADDING_IDX)

            @pl.when(has_pad)
            def _():
                idx_v[...] = jnp.where(
                    tok != PADDING_IDX, s0 + j * LANES + lane + 1, PADDING_IDX
                )
                for q in range(NQ):
                    pltpu.async_copy(
                        w_hbm.at[idx_v.at[pl.ds(q * QROWS, QROWS)]],
                        bufs[q],
                        semgs[q],
                    ).wait()
                    pltpu.sync_copy(
                        bufs[q],
                        out_hbm.at[
                            pl.ds(b * seq_len + s0 + j * LANES + q * QROWS, QROWS)
                        ],
                    )

        for b in range(bsz):
            lax.fori_loop(0, n_groups, lambda j, c, b=b: (fix_group(b, j), c)[1], 0)

    return sc_embed


def kernel(input, weights):
    bsz, seq_len = input.shape
    d = weights.shape[1]
    sc_embed = _make_sc_embed(bsz, seq_len, d)
    out = sc_embed(input.reshape(-1), weights)
    return out.reshape(bsz, seq_len, d)
